# baseline (speedup 1.0000x reference)
import functools

import jax
import jax.numpy as jnp
from jax import lax
from jax.experimental import pallas as pl
from jax.experimental.pallas import tpu as pltpu, tpu_sc as plsc

_NQ = 2048
_NW = 32
_QPT = _NQ // _NW
_STAGE_C = (144, 288, 576, 1152, 2304)
_STAGE_CP = (256, 384, 640, 1152, 2304)
_STAGE_CQ = (32, 32, 32, 16, 8)


def _topk_stage_body(q_ref, b_ref, x_ref, oi_ref, ow_ref, *, two_n):
    n = two_n // 2
    qb = q_ref.shape[0]
    X = x_ref[...]
    sumx = jnp.sum(X * X, axis=0, keepdims=True)
    qv = q_ref[:, 0:3]
    sumq = jnp.sum(qv * qv, axis=1, keepdims=True)
    Xb = X.astype(jnp.bfloat16).astype(jnp.float32)
    qb16 = qv.astype(jnp.bfloat16).astype(jnp.float32)
    cross = (qb16[:, 0:1] * Xb[0:1, :]
             + qb16[:, 1:2] * Xb[1:2, :]
             + qb16[:, 2:3] * Xb[2:3, :])
    d2 = sumq + sumx - 2.0 * cross
    lane = lax.broadcasted_iota(jnp.int32, (qb, two_n), 1)
    lane_batch = (lane >= n).astype(jnp.int32)
    bq = b_ref[:, 0:1]
    d2 = jnp.where(lane_batch == bq, d2, jnp.inf)

    vals, idxs = [], []
    for k in range(3):
        m = jnp.min(d2, axis=1, keepdims=True)
        im = jnp.min(jnp.where(d2 == m, lane, two_n), axis=1, keepdims=True)
        vals.append(m)
        idxs.append(im)
        if k < 2:
            d2 = jnp.where(lane == im, jnp.inf, d2)

    rec = [1.0 / (jnp.maximum(v, 1e-10) + 1e-8) for v in vals]
    rsum = rec[0] + rec[1] + rec[2]
    w = [r / rsum for r in rec]
    zi = jnp.zeros((qb, 128 - 3), jnp.int32)
    oi_ref[...] = jnp.concatenate(idxs + [zi], axis=1)
    zw = jnp.zeros((qb, 128 - 3), jnp.float32)
    ow_ref[...] = jnp.concatenate(w + [zw], axis=1)


def _topk_stage(qpad, bpad, xyzT, qb):
    two_n = xyzT.shape[1]
    body = functools.partial(_topk_stage_body, two_n=two_n)
    body = functools.wraps(_topk_stage_body)(body)
    body.__name__ = f"topk_n{two_n}_q{qb}"
    return pl.pallas_call(
        body,
        grid=(_NQ // qb,),
        in_specs=[
            pl.BlockSpec((qb, 128), lambda i: (i, 0)),
            pl.BlockSpec((qb, 128), lambda i: (i, 0)),
            pl.BlockSpec((3, two_n), lambda i: (0, 0)),
        ],
        out_specs=[
            pl.BlockSpec((qb, 128), lambda i: (i, 0)),
            pl.BlockSpec((qb, 128), lambda i: (i, 0)),
        ],
        out_shape=[
            jax.ShapeDtypeStruct((_NQ, 128), jnp.int32),
            jax.ShapeDtypeStruct((_NQ, 128), jnp.float32),
        ],
    )(qpad, bpad, xyzT)


def _sc_body(*refs):
    ins = refs[:15]
    outs = refs[15:20]
    sem = refs[20]
    wid = lax.axis_index("s") * 2 + lax.axis_index("c")
    base = wid * _QPT

    for s in range(5):
        table, ih, wh = ins[3 * s], ins[3 * s + 1], ins[3 * s + 2]
        out = outs[s]
        c = _STAGE_CP[s]
        cq = _STAGE_CQ[s]
        nch = _QPT // cq

        def stage_scope(idxv, wv, rows, outb, table=table, ih=ih, wh=wh,
                        out=out, c=c, cq=cq, nch=nch):
            for ci in range(nch):
                ioff = base * 3 + ci * 3 * cq
                qoff = base + ci * cq
                pltpu.sync_copy(ih.at[pl.ds(ioff, 3 * cq)], idxv)
                pltpu.sync_copy(wh.at[pl.ds(ioff, 3 * cq)], wv)
                pltpu.async_copy(table.at[idxv], rows, sem).wait()
                for q in range(cq):
                    w0 = wv[3 * q, :]
                    w1 = wv[3 * q + 1, :]
                    w2 = wv[3 * q + 2, :]

                    def body(j, _, q=q, w0=w0, w1=w1, w2=w2):
                        sl = pl.ds(j * 16, 16)
                        outb[q, sl] = (w0 * rows[3 * q, sl]
                                       + w1 * rows[3 * q + 1, sl]
                                       + w2 * rows[3 * q + 2, sl])
                        return 0

                    lax.fori_loop(0, c // 16, body, 0)
                pltpu.sync_copy(outb, out.at[pl.ds(qoff, cq)])

        pl.run_scoped(
            stage_scope,
            pltpu.VMEM((3 * cq,), jnp.int32),
            pltpu.VMEM((3 * cq, 16), jnp.float32),
            pltpu.VMEM((3 * cq, c), jnp.float32),
            pltpu.VMEM((cq, c), jnp.float32),
        )


def _sc_gather(tables, idxs, wexps):
    mesh = plsc.VectorSubcoreMesh(core_axis_name="c", subcore_axis_name="s")
    k = pl.kernel(
        _sc_body,
        mesh=mesh,
        out_type=[jax.ShapeDtypeStruct((_NQ, c), jnp.float32)
                  for c in _STAGE_CP],
        scratch_types=[pltpu.SemaphoreType.DMA],
    )
    args = []
    for t, i, w in zip(tables, idxs, wexps):
        args += [t, i, w]
    return k(*args)


def _mlp_body(x_ref, w1_ref, w2_ref, w3_ref, w4_ref, b4_ref, o_ref):
    def mm(a, b):
        return lax.dot_general(
            a.astype(jnp.bfloat16), b.astype(jnp.bfloat16),
            dimension_numbers=(((1,), (1,)), ((), ())),
            preferred_element_type=jnp.float32)
    h = jax.nn.relu(mm(x_ref[...], w1_ref[...]))
    h = jax.nn.relu(mm(h, w2_ref[...]))
    h = jax.nn.relu(mm(h, w3_ref[...]))
    o_ref[...] = mm(h, w4_ref[...]) + b4_ref[0:1, :]


def _mlp(x, W1p, W2, W3p, W4pp, b4p, qb):
    width = x.shape[1]
    h1, h2 = W1p.shape[0], W2.shape[0]
    h3p = W3p.shape[0]
    return pl.pallas_call(
        _mlp_body,
        grid=(_NQ // qb,),
        in_specs=[
            pl.BlockSpec((qb, width), lambda i: (i, 0)),
            pl.BlockSpec((h1, width), lambda i: (0, 0)),
            pl.BlockSpec((h2, h1), lambda i: (0, 0)),
            pl.BlockSpec((h3p, h2), lambda i: (0, 0)),
            pl.BlockSpec((h3p, h3p), lambda i: (0, 0)),
            pl.BlockSpec((8, h3p), lambda i: (0, 0)),
        ],
        out_specs=pl.BlockSpec((qb, h3p), lambda i: (i, 0)),
        out_shape=jax.ShapeDtypeStruct((_NQ, h3p), jnp.float32),
    )(x, W1p, W2, W3p, W4pp, b4p)


def kernel(weakly_points, res1_xyz, res1_features, res2_xyz, res2_features,
           res3_xyz, res3_features, res4_xyz, res4_features, res5_xyz,
           res5_features, batch_inds, W1, W2, W3, W4, b4):
    qpad = jnp.pad(weakly_points, ((0, 0), (0, 125)))
    bpad = jnp.pad(batch_inds.reshape(-1, 1), ((0, 0), (0, 127)))
    stages = [(res1_xyz, res1_features), (res2_xyz, res2_features),
              (res3_xyz, res3_features), (res4_xyz, res4_features),
              (res5_xyz, res5_features)]

    tables, idx_flat, wexp = [], [], []
    for s, ((xyz, feat), qb) in enumerate(zip(stages,
                                              (128, 256, 512, 512, 512))):
        xyzT = jnp.concatenate([xyz[0], xyz[1]], axis=0).T
        oi, ow = _topk_stage(qpad, bpad, xyzT, qb)
        idx_flat.append(oi[:, :3].reshape(-1))
        wexp.append(jnp.broadcast_to(ow[:, :3].reshape(-1)[:, None],
                                     (3 * _NQ, 16)))
        table = jnp.concatenate([feat[0].T, feat[1].T], axis=0)
        tables.append(jnp.pad(table,
                              ((0, 0), (0, _STAGE_CP[s] - _STAGE_C[s]))))

    feats = _sc_gather(tables, idx_flat, wexp)
    x = jnp.concatenate(feats, axis=1)

    w1_cols = []
    off = 0
    for c, cp in zip(_STAGE_C, _STAGE_CP):
        blk = W1[:, off:off + c]
        w1_cols.append(jnp.pad(blk, ((0, 0), (0, cp - c))))
        off += c
    W1p = jnp.concatenate(w1_cols, axis=1)

    h3p = 128
    W3p = jnp.pad(W3, ((0, h3p - W3.shape[0]), (0, 0)))
    W4pp = jnp.pad(W4, ((0, h3p - W4.shape[0]),
                        (0, h3p - W4.shape[1])))
    b4p = jnp.broadcast_to(jnp.pad(b4, (0, h3p - b4.shape[0]))[None, :],
                           (8, h3p))
    out = _mlp(x, W1p, W2, W3p, W4pp, b4p, 256)
    return out[:, :W4.shape[0]]


# speedup vs baseline: 1.0821x; 1.0821x over previous
import functools

import jax
import jax.numpy as jnp
from jax import lax
from jax.experimental import pallas as pl
from jax.experimental.pallas import tpu as pltpu, tpu_sc as plsc

_NQ = 2048
_NW = 32
_QPT = _NQ // _NW
_STAGE_C = (144, 288, 576, 1152, 2304)
_STAGE_CP = (256, 384, 640, 1152, 2304)
_STAGE_CQ = (32, 32, 32, 16, 8)


def _topk_stage_body(q_ref, b_ref, x_ref, oi_ref, ow_ref, *, two_n):
    n = two_n // 2
    qb = q_ref.shape[0]
    X = x_ref[...]
    sumx = jnp.sum(X * X, axis=0, keepdims=True)
    qv = q_ref[:, 0:3]
    sumq = jnp.sum(qv * qv, axis=1, keepdims=True)
    Xb = X.astype(jnp.bfloat16).astype(jnp.float32)
    qb16 = qv.astype(jnp.bfloat16).astype(jnp.float32)
    cross = (qb16[:, 0:1] * Xb[0:1, :]
             + qb16[:, 1:2] * Xb[1:2, :]
             + qb16[:, 2:3] * Xb[2:3, :])
    d2 = sumq + sumx - 2.0 * cross
    lane = lax.broadcasted_iota(jnp.int32, (qb, two_n), 1)
    lane_batch = (lane >= n).astype(jnp.int32)
    bq = b_ref[:, 0:1]
    d2 = jnp.where(lane_batch == bq, d2, jnp.inf)

    vals, idxs = [], []
    for k in range(3):
        m = jnp.min(d2, axis=1, keepdims=True)
        im = jnp.min(jnp.where(d2 == m, lane, two_n), axis=1, keepdims=True)
        vals.append(m)
        idxs.append(im)
        if k < 2:
            d2 = jnp.where(lane == im, jnp.inf, d2)

    rec = [1.0 / (jnp.maximum(v, 1e-10) + 1e-8) for v in vals]
    rsum = rec[0] + rec[1] + rec[2]
    w = [r / rsum for r in rec]
    zi = jnp.zeros((qb, 128 - 3), jnp.int32)
    oi_ref[...] = jnp.concatenate(idxs + [zi], axis=1)
    zw = jnp.zeros((qb, 128 - 3), jnp.float32)
    ow_ref[...] = jnp.concatenate(w + [zw], axis=1)


def _topk_stage(qpad, bpad, xyzT, qb):
    two_n = xyzT.shape[1]
    body = functools.partial(_topk_stage_body, two_n=two_n)
    body = functools.wraps(_topk_stage_body)(body)
    body.__name__ = f"topk_n{two_n}_q{qb}"
    return pl.pallas_call(
        body,
        grid=(_NQ // qb,),
        in_specs=[
            pl.BlockSpec((qb, 128), lambda i: (i, 0)),
            pl.BlockSpec((qb, 128), lambda i: (i, 0)),
            pl.BlockSpec((3, two_n), lambda i: (0, 0)),
        ],
        out_specs=[
            pl.BlockSpec((qb, 128), lambda i: (i, 0)),
            pl.BlockSpec((qb, 128), lambda i: (i, 0)),
        ],
        out_shape=[
            jax.ShapeDtypeStruct((_NQ, 128), jnp.int32),
            jax.ShapeDtypeStruct((_NQ, 128), jnp.float32),
        ],
    )(qpad, bpad, xyzT)


def _sc_stage_body(table, ih, wh, out, sem, *, s):
    wid = lax.axis_index("s") * 2 + lax.axis_index("c")
    base = wid * _QPT
    c = _STAGE_CP[s]
    cq = _STAGE_CQ[s]
    nch = _QPT // cq

    def stage_scope(idxv, wv, rows, outb):
        for ci in range(nch):
            ioff = base * 3 + ci * 3 * cq
            qoff = base + ci * cq
            pltpu.sync_copy(ih.at[pl.ds(ioff, 3 * cq)], idxv)
            pltpu.sync_copy(wh.at[pl.ds(ioff, 3 * cq)], wv)
            pltpu.async_copy(table.at[idxv], rows, sem).wait()
            for q in range(cq):
                w0 = wv[3 * q, :]
                w1 = wv[3 * q + 1, :]
                w2 = wv[3 * q + 2, :]

                def body(j, _, q=q, w0=w0, w1=w1, w2=w2):
                    sl = pl.ds(j * 16, 16)
                    outb[q, sl] = (w0 * rows[3 * q, sl]
                                   + w1 * rows[3 * q + 1, sl]
                                   + w2 * rows[3 * q + 2, sl])
                    return 0

                lax.fori_loop(0, c // 16, body, 0)
            pltpu.sync_copy(outb, out.at[pl.ds(qoff, cq)])

    pl.run_scoped(
        stage_scope,
        pltpu.VMEM((3 * cq,), jnp.int32),
        pltpu.VMEM((3 * cq, 16), jnp.float32),
        pltpu.VMEM((3 * cq, c), jnp.float32),
        pltpu.VMEM((cq, c), jnp.float32),
    )


def _sc_gather_stage(table, idx, wexp, s):
    mesh = plsc.VectorSubcoreMesh(core_axis_name="c", subcore_axis_name="s")
    body = functools.partial(_sc_stage_body, s=s)
    body = functools.wraps(_sc_stage_body)(body)
    body.__name__ = f"sc_gather_s{s}"
    k = pl.kernel(
        body,
        mesh=mesh,
        out_type=jax.ShapeDtypeStruct((_NQ, _STAGE_CP[s]), jnp.float32),
        scratch_types=[pltpu.SemaphoreType.DMA],
    )
    return k(table, idx, wexp)


def _mlp_body(x_ref, w1_ref, w2_ref, w3_ref, w4_ref, b4_ref, o_ref):
    def mm(a, b):
        return lax.dot_general(
            a.astype(jnp.bfloat16), b.astype(jnp.bfloat16),
            dimension_numbers=(((1,), (1,)), ((), ())),
            preferred_element_type=jnp.float32)
    h = jax.nn.relu(mm(x_ref[...], w1_ref[...]))
    h = jax.nn.relu(mm(h, w2_ref[...]))
    h = jax.nn.relu(mm(h, w3_ref[...]))
    o_ref[...] = mm(h, w4_ref[...]) + b4_ref[0:1, :]


def _mlp(x, W1p, W2, W3p, W4pp, b4p, qb):
    width = x.shape[1]
    h1, h2 = W1p.shape[0], W2.shape[0]
    h3p = W3p.shape[0]
    return pl.pallas_call(
        _mlp_body,
        grid=(_NQ // qb,),
        in_specs=[
            pl.BlockSpec((qb, width), lambda i: (i, 0)),
            pl.BlockSpec((h1, width), lambda i: (0, 0)),
            pl.BlockSpec((h2, h1), lambda i: (0, 0)),
            pl.BlockSpec((h3p, h2), lambda i: (0, 0)),
            pl.BlockSpec((h3p, h3p), lambda i: (0, 0)),
            pl.BlockSpec((8, h3p), lambda i: (0, 0)),
        ],
        out_specs=pl.BlockSpec((qb, h3p), lambda i: (i, 0)),
        out_shape=jax.ShapeDtypeStruct((_NQ, h3p), jnp.float32),
    )(x, W1p, W2, W3p, W4pp, b4p)


def kernel(weakly_points, res1_xyz, res1_features, res2_xyz, res2_features,
           res3_xyz, res3_features, res4_xyz, res4_features, res5_xyz,
           res5_features, batch_inds, W1, W2, W3, W4, b4):
    qpad = jnp.pad(weakly_points, ((0, 0), (0, 125)))
    bpad = jnp.pad(batch_inds.reshape(-1, 1), ((0, 0), (0, 127)))
    stages = [(res1_xyz, res1_features), (res2_xyz, res2_features),
              (res3_xyz, res3_features), (res4_xyz, res4_features),
              (res5_xyz, res5_features)]

    feats = []
    for s, ((xyz, feat), qb) in enumerate(zip(stages,
                                              (128, 256, 512, 512, 512))):
        xyzT = jnp.concatenate([xyz[0], xyz[1]], axis=0).T
        oi, ow = _topk_stage(qpad, bpad, xyzT, qb)
        idx_flat = oi[:, :3].reshape(-1)
        wexp = jnp.broadcast_to(ow[:, :3].reshape(-1)[:, None],
                                (3 * _NQ, 16))
        table = jnp.concatenate([feat[0].T, feat[1].T], axis=0)
        table = jnp.pad(table, ((0, 0), (0, _STAGE_CP[s] - _STAGE_C[s])))
        feats.append(_sc_gather_stage(table, idx_flat, wexp, s))

    x = jnp.concatenate(feats, axis=1)

    w1_cols = []
    off = 0
    for c, cp in zip(_STAGE_C, _STAGE_CP):
        blk = W1[:, off:off + c]
        w1_cols.append(jnp.pad(blk, ((0, 0), (0, cp - c))))
        off += c
    W1p = jnp.concatenate(w1_cols, axis=1)

    h3p = 128
    W3p = jnp.pad(W3, ((0, h3p - W3.shape[0]), (0, 0)))
    W4pp = jnp.pad(W4, ((0, h3p - W4.shape[0]),
                        (0, h3p - W4.shape[1])))
    b4p = jnp.broadcast_to(jnp.pad(b4, (0, h3p - b4.shape[0]))[None, :],
                           (8, h3p))
    out = _mlp(x, W1p, W2, W3p, W4pp, b4p, 256)
    return out[:, :W4.shape[0]]


# speedup vs baseline: 1.0999x; 1.0165x over previous
import functools

import jax
import jax.numpy as jnp
from jax import lax
from jax.experimental import pallas as pl
from jax.experimental.pallas import tpu as pltpu, tpu_sc as plsc

_NQ = 2048
_NW = 32
_QPT = _NQ // _NW
_STAGE_C = (144, 288, 576, 1152, 2304)
_STAGE_CP = (256, 384, 640, 1152, 2304)
_STAGE_CQ = (32, 32, 32, 16, 8)


def _topk_stage_body(q_ref, b_ref, x_ref, oi_ref, ow_ref, *, two_n):
    n = two_n // 2
    qb = q_ref.shape[0]
    X = x_ref[...]
    sumx = jnp.sum(X * X, axis=0, keepdims=True)
    qv = q_ref[:, 0:3]
    sumq = jnp.sum(qv * qv, axis=1, keepdims=True)
    Xb = X.astype(jnp.bfloat16).astype(jnp.float32)
    qb16 = qv.astype(jnp.bfloat16).astype(jnp.float32)
    cross = (qb16[:, 0:1] * Xb[0:1, :]
             + qb16[:, 1:2] * Xb[1:2, :]
             + qb16[:, 2:3] * Xb[2:3, :])
    d2 = sumq + sumx - 2.0 * cross
    lane = lax.broadcasted_iota(jnp.int32, (qb, two_n), 1)
    lane_batch = (lane >= n).astype(jnp.int32)
    bq = b_ref[:, 0:1]
    d2 = jnp.where(lane_batch == bq, d2, jnp.inf)

    vals, idxs = [], []
    for k in range(3):
        m = jnp.min(d2, axis=1, keepdims=True)
        im = jnp.min(jnp.where(d2 == m, lane, two_n), axis=1, keepdims=True)
        vals.append(m)
        idxs.append(im)
        if k < 2:
            d2 = jnp.where(lane == im, jnp.inf, d2)

    rec = [1.0 / (jnp.maximum(v, 1e-10) + 1e-8) for v in vals]
    rsum = rec[0] + rec[1] + rec[2]
    w = [r / rsum for r in rec]
    zi = jnp.zeros((qb, 128 - 3), jnp.int32)
    oi_ref[...] = jnp.concatenate(idxs + [zi], axis=1)
    zw = jnp.zeros((qb, 128 - 3), jnp.float32)
    ow_ref[...] = jnp.concatenate(w + [zw], axis=1)


def _topk_stage(qpad, bpad, xyzT, qb):
    two_n = xyzT.shape[1]
    body = functools.partial(_topk_stage_body, two_n=two_n)
    body = functools.wraps(_topk_stage_body)(body)
    body.__name__ = f"topk_n{two_n}_q{qb}"
    return pl.pallas_call(
        body,
        grid=(_NQ // qb,),
        in_specs=[
            pl.BlockSpec((qb, 128), lambda i: (i, 0)),
            pl.BlockSpec((qb, 128), lambda i: (i, 0)),
            pl.BlockSpec((3, two_n), lambda i: (0, 0)),
        ],
        out_specs=[
            pl.BlockSpec((qb, 128), lambda i: (i, 0)),
            pl.BlockSpec((qb, 128), lambda i: (i, 0)),
        ],
        out_shape=[
            jax.ShapeDtypeStruct((_NQ, 128), jnp.int32),
            jax.ShapeDtypeStruct((_NQ, 128), jnp.float32),
        ],
    )(qpad, bpad, xyzT)


def _sc_stage_body(table, ih, wh, out, sem, *, s):
    wid = lax.axis_index("s") * 2 + lax.axis_index("c")
    base = wid * _QPT
    c = _STAGE_CP[s]
    cq = _STAGE_CQ[s]
    nch = _QPT // cq

    def stage_scope(idxv, wv, rows, outb):
        for ci in range(nch):
            ioff = base * 3 + ci * 3 * cq
            qoff = base + ci * cq
            pltpu.sync_copy(ih.at[pl.ds(ioff, 3 * cq)], idxv)
            pltpu.sync_copy(wh.at[pl.ds(ioff, 3 * cq)], wv)
            pltpu.async_copy(table.at[idxv], rows, sem).wait()
            for q in range(cq):
                w0 = wv[3 * q, :]
                w1 = wv[3 * q + 1, :]
                w2 = wv[3 * q + 2, :]

                def body(j, _, q=q, w0=w0, w1=w1, w2=w2):
                    sl = pl.ds(j * 16, 16)
                    outb[q, sl] = (w0 * rows[3 * q, sl]
                                   + w1 * rows[3 * q + 1, sl]
                                   + w2 * rows[3 * q + 2, sl])
                    return 0

                lax.fori_loop(0, c // 16, body, 0)
            pltpu.sync_copy(outb, out.at[pl.ds(qoff, cq)])

    pl.run_scoped(
        stage_scope,
        pltpu.VMEM((3 * cq,), jnp.int32),
        pltpu.VMEM((3 * cq, 16), jnp.float32),
        pltpu.VMEM((3 * cq, c), jnp.float32),
        pltpu.VMEM((cq, c), jnp.float32),
    )


def _sc_gather_stage(table, idx, wexp, s):
    mesh = plsc.VectorSubcoreMesh(core_axis_name="c", subcore_axis_name="s")
    body = functools.partial(_sc_stage_body, s=s)
    body = functools.wraps(_sc_stage_body)(body)
    body.__name__ = f"sc_gather_s{s}"
    k = pl.kernel(
        body,
        mesh=mesh,
        out_type=jax.ShapeDtypeStruct((_NQ, _STAGE_CP[s]), jnp.float32),
        scratch_types=[pltpu.SemaphoreType.DMA],
    )
    return k(table, idx, wexp)


def _mlp_body(x_ref, w1_ref, w2_ref, w3_ref, w4_ref, b4_ref, o_ref):
    def mm(a, b):
        return lax.dot_general(
            a.astype(jnp.bfloat16), b.astype(jnp.bfloat16),
            dimension_numbers=(((1,), (1,)), ((), ())),
            preferred_element_type=jnp.float32)
    h = jax.nn.relu(mm(x_ref[...], w1_ref[...]))
    h = jax.nn.relu(mm(h, w2_ref[...]))
    h = jax.nn.relu(mm(h, w3_ref[...]))
    o_ref[...] = mm(h, w4_ref[...]) + b4_ref[0:1, :]


def _mlp(x, W1p, W2, W3p, W4pp, b4p, qb):
    width = x.shape[1]
    h1, h2 = W1p.shape[0], W2.shape[0]
    h3p = W3p.shape[0]
    return pl.pallas_call(
        _mlp_body,
        grid=(_NQ // qb,),
        in_specs=[
            pl.BlockSpec((qb, width), lambda i: (i, 0)),
            pl.BlockSpec((h1, width), lambda i: (0, 0)),
            pl.BlockSpec((h2, h1), lambda i: (0, 0)),
            pl.BlockSpec((h3p, h2), lambda i: (0, 0)),
            pl.BlockSpec((h3p, h3p), lambda i: (0, 0)),
            pl.BlockSpec((8, h3p), lambda i: (0, 0)),
        ],
        out_specs=pl.BlockSpec((qb, h3p), lambda i: (i, 0)),
        out_shape=jax.ShapeDtypeStruct((_NQ, h3p), jnp.float32),
    )(x, W1p, W2, W3p, W4pp, b4p)


def kernel(weakly_points, res1_xyz, res1_features, res2_xyz, res2_features,
           res3_xyz, res3_features, res4_xyz, res4_features, res5_xyz,
           res5_features, batch_inds, W1, W2, W3, W4, b4):
    qpad = jnp.pad(weakly_points, ((0, 0), (0, 125)))
    bpad = jnp.pad(batch_inds.reshape(-1, 1), ((0, 0), (0, 127)))
    stages = [(res1_xyz, res1_features), (res2_xyz, res2_features),
              (res3_xyz, res3_features), (res4_xyz, res4_features),
              (res5_xyz, res5_features)]

    feats = []
    for s, ((xyz, feat), qb) in enumerate(zip(stages,
                                              (256, 512, 512, 512, 512))):
        xyzT = jnp.concatenate([xyz[0], xyz[1]], axis=0).T
        oi, ow = _topk_stage(qpad, bpad, xyzT, qb)
        idx_flat = oi[:, :3].reshape(-1)
        wexp = jnp.broadcast_to(ow[:, :3].reshape(-1)[:, None],
                                (3 * _NQ, 16))
        table = jnp.concatenate([feat[0].T, feat[1].T], axis=0)
        table = jnp.pad(table, ((0, 0), (0, _STAGE_CP[s] - _STAGE_C[s])))
        feats.append(_sc_gather_stage(table, idx_flat, wexp, s))

    x = jnp.concatenate(feats, axis=1)

    w1_cols = []
    off = 0
    for c, cp in zip(_STAGE_C, _STAGE_CP):
        blk = W1[:, off:off + c]
        w1_cols.append(jnp.pad(blk, ((0, 0), (0, cp - c))))
        off += c
    W1p = jnp.concatenate(w1_cols, axis=1)

    h3p = 128
    W3p = jnp.pad(W3, ((0, h3p - W3.shape[0]), (0, 0)))
    W4pp = jnp.pad(W4, ((0, h3p - W4.shape[0]),
                        (0, h3p - W4.shape[1])))
    b4p = jnp.broadcast_to(jnp.pad(b4, (0, h3p - b4.shape[0]))[None, :],
                           (8, h3p))
    out = _mlp(x, W1p, W2, W3p, W4pp, b4p, 256)
    return out[:, :W4.shape[0]]
